# baseline (speedup 1.0000x reference)
import functools

import jax
import jax.numpy as jnp
from jax import lax
from jax.experimental import pallas as pl
from jax.experimental.pallas import tpu as pltpu
from jax.experimental.pallas import tpu_sc as plsc

N, H, W, C = 2, 384, 384, 96
HO, WO = 384, 384
P = N * HO * WO
NHW = N * H * W


_K1_ROWS = 16
_K1_T = H // _K1_ROWS


CP = 128


def _theta_body(x_ref, w_ref, b_ref, theta_ref, acc_ref):
    t = pl.program_id(0)

    @pl.when(t == 0)
    def _():
        acc_ref[...] = jnp.zeros_like(acc_ref)

    acc_ref[...] += jnp.sum(x_ref[...], axis=(1, 2))

    @pl.when(t == _K1_T - 1)
    def _():
        mean = acc_ref[...] * (1.0 / (H * W))
        theta_ref[...] = (
            jax.lax.dot(mean, w_ref[...],
                        preferred_element_type=jnp.float32)
            + b_ref[...]
        )


def _compute_theta(x, w_loc, b_loc):
    return pl.pallas_call(
        _theta_body,
        grid=(_K1_T,),
        in_specs=[
            pl.BlockSpec((N, _K1_ROWS, W, C), lambda t: (0, t, 0, 0)),
            pl.BlockSpec((C, 6), lambda t: (0, 0)),
            pl.BlockSpec((1, 6), lambda t: (0, 0)),
        ],
        out_specs=pl.BlockSpec((N, 6), lambda t: (0, 0)),
        out_shape=jax.ShapeDtypeStruct((N, 6), jnp.float32),
        scratch_shapes=[pltpu.VMEM((N, C), jnp.float32)],
    )(x, w_loc, b_loc.reshape(1, 6))


_K2_ROWS = 64
_K2_T = HO // _K2_ROWS


def _idxw_body(theta_ref, idx_ref, w_ref):
    t = pl.program_id(0)

    ii = (lax.broadcasted_iota(jnp.int32, (_K2_ROWS, WO), 0)
          + t * _K2_ROWS).astype(jnp.float32)
    jj = lax.broadcasted_iota(jnp.int32, (_K2_ROWS, WO), 1).astype(jnp.float32)
    def _b(v):
        return v.astype(jnp.bfloat16).astype(jnp.float32)

    gy = _b(ii / (HO - 1.0) * 2.0 - 1.0)
    gx = _b(jj / (WO - 1.0) * 2.0 - 1.0)

    def _bs(s):
        return _b(jnp.full((_K2_ROWS, WO), s, jnp.float32))

    for n in range(N):
        t00 = _bs(theta_ref[n, 0])
        t01 = _bs(theta_ref[n, 1])
        t02 = _bs(theta_ref[n, 2])
        t10 = _bs(theta_ref[n, 3])
        t11 = _bs(theta_ref[n, 4])
        t12 = _bs(theta_ref[n, 5])
        yc = (t00 * gy + t01 * gx + t02 + 1.0) * ((H - 1) / 2.0)
        xc = (t10 * gy + t11 * gx + t12 + 1.0) * ((W - 1) / 2.0)
        y0 = jnp.floor(yc)
        x0 = jnp.floor(xc)
        wy1 = 1.0 - jnp.abs(y0 + 1.0 - yc)
        wy0 = 1.0 - jnp.abs(y0 - yc)
        wx1 = 1.0 - jnp.abs(x0 + 1.0 - xc)
        wx0 = 1.0 - jnp.abs(x0 - xc)
        y0c = jnp.clip(y0, 0.0, H - 1.0).astype(jnp.int32)
        y1c = jnp.clip(y0 + 1.0, 0.0, H - 1.0).astype(jnp.int32)
        x0c = jnp.clip(x0, 0.0, W - 1.0).astype(jnp.int32)
        x1c = jnp.clip(x0 + 1.0, 0.0, W - 1.0).astype(jnp.int32)
        r0 = n * (H * W) + y0c * W
        r1 = n * (H * W) + y1c * W
        idx_ref[0, n] = r0 + x0c
        idx_ref[1, n] = r0 + x1c
        idx_ref[2, n] = r1 + x0c
        idx_ref[3, n] = r1 + x1c
        w_ref[0, n] = wy0 * wx0
        w_ref[1, n] = wy0 * wx1
        w_ref[2, n] = wy1 * wx0
        w_ref[3, n] = wy1 * wx1


def _compute_idx_w(theta):
    return pl.pallas_call(
        _idxw_body,
        grid=(_K2_T,),
        in_specs=[pl.BlockSpec(memory_space=pltpu.SMEM)],
        out_specs=[
            pl.BlockSpec((4, N, _K2_ROWS, WO), lambda t: (0, 0, t, 0)),
            pl.BlockSpec((4, N, _K2_ROWS, WO), lambda t: (0, 0, t, 0)),
        ],
        out_shape=[
            jax.ShapeDtypeStruct((4, N, HO, WO), jnp.int32),
            jax.ShapeDtypeStruct((4, N, HO, WO), jnp.float32),
        ],
    )(theta)


_GW = 128


def _sc_gather(x_flat, idx):
    mesh = plsc.VectorSubcoreMesh(core_axis_name="c", subcore_axis_name="s")
    n_idx = idx.shape[0]
    cw = x_flat.shape[1]

    @functools.partial(
        pl.kernel,
        out_type=jax.ShapeDtypeStruct((n_idx, cw), jnp.float32),
        mesh=mesh,
        compiler_params=pltpu.CompilerParams(use_tc_tiling_on_sc=False),
    )
    def gather_kernel(x_hbm, i_hbm, o_hbm):
        def body(i_vmem, o_vmem):
            pltpu.sync_copy(x_hbm.at[i_vmem], o_vmem)

        pltpu.emit_pipeline(
            body,
            grid=(n_idx // _GW,),
            in_specs=[pl.BlockSpec((_GW,), lambda i: (i,))],
            out_specs=[pl.BlockSpec((_GW, cw), lambda i: (i, 0))],
            core_axis_name=("c", "s"),
            dimension_semantics=(pltpu.PARALLEL,),
        )(i_hbm, o_hbm)

    return gather_kernel(x_flat, idx)


_K4_BP = 1024
_K4_T = P // _K4_BP


def _wsum_body(g_ref, w_ref, o_ref):
    o_ref[...] = (
        w_ref[0].reshape(_K4_BP, 1) * g_ref[0]
        + w_ref[1].reshape(_K4_BP, 1) * g_ref[1]
        + w_ref[2].reshape(_K4_BP, 1) * g_ref[2]
        + w_ref[3].reshape(_K4_BP, 1) * g_ref[3]
    )


def _weighted_sum(g, w_flat):
    return pl.pallas_call(
        _wsum_body,
        grid=(_K4_T,),
        in_specs=[
            pl.BlockSpec((4, _K4_BP, C), lambda t: (0, t, 0)),
            pl.BlockSpec((4, _K4_BP), lambda t: (0, t)),
        ],
        out_specs=pl.BlockSpec((_K4_BP, C), lambda t: (t, 0)),
        out_shape=jax.ShapeDtypeStruct((P, C), jnp.float32),
    )(g, w_flat)


def kernel(x, W_loc, b_loc):
    theta = _compute_theta(x, W_loc, b_loc)
    idx, w = _compute_idx_w(theta)
    g = _sc_gather(x.reshape(NHW, C), idx.reshape(4 * P))
    out = _weighted_sum(g.reshape(4, P, C), w.reshape(4, P))
    return out.reshape(N, HO, WO, C)


# speedup vs baseline: 1.5198x; 1.5198x over previous
import functools

import jax
import jax.numpy as jnp
from jax import lax
from jax.experimental import pallas as pl
from jax.experimental.pallas import tpu as pltpu
from jax.experimental.pallas import tpu_sc as plsc

N, H, W, C = 2, 384, 384, 96
HO, WO = 384, 384
P = N * HO * WO
NHW = N * H * W


_K1_ROWS = 16
_K1_T = H // _K1_ROWS


CP = 128
CP2 = 256


def _theta_body(x_ref, w_ref, b_ref, theta_ref, xpad_ref, acc_ref):
    t = pl.program_id(0)

    @pl.when(t == 0)
    def _():
        acc_ref[...] = jnp.zeros_like(acc_ref)

    xb = x_ref[...]
    acc_ref[...] += jnp.sum(xb, axis=(1, 2))
    xpad_ref[:, :, :, 0:C] = xb
    xpad_ref[:, :, :, CP:CP + C] = jnp.concatenate(
        [xb[:, :, 1:, :], xb[:, :, W - 1:, :]], axis=2)

    @pl.when(t == _K1_T - 1)
    def _():
        mean = acc_ref[...] * (1.0 / (H * W))
        theta_ref[...] = (
            jax.lax.dot(mean, w_ref[...],
                        preferred_element_type=jnp.float32)
            + b_ref[...]
        )


def _compute_theta(x, w_loc, b_loc):
    return pl.pallas_call(
        _theta_body,
        grid=(_K1_T,),
        in_specs=[
            pl.BlockSpec((N, _K1_ROWS, W, C), lambda t: (0, t, 0, 0)),
            pl.BlockSpec((C, 6), lambda t: (0, 0)),
            pl.BlockSpec((1, 6), lambda t: (0, 0)),
        ],
        out_specs=[
            pl.BlockSpec((N, 6), lambda t: (0, 0)),
            pl.BlockSpec((N, _K1_ROWS, W, CP2), lambda t: (0, t, 0, 0)),
        ],
        out_shape=[
            jax.ShapeDtypeStruct((N, 6), jnp.float32),
            jax.ShapeDtypeStruct((N, H, W, CP2), jnp.float32),
        ],
        scratch_shapes=[pltpu.VMEM((N, C), jnp.float32)],
    )(x, w_loc, b_loc.reshape(1, 6))


_K2_ROWS = 64
_K2_T = HO // _K2_ROWS


def _idxw_body(theta_ref, idx_ref, w_ref):
    t = pl.program_id(0)

    ii = (lax.broadcasted_iota(jnp.int32, (_K2_ROWS, WO), 0)
          + t * _K2_ROWS).astype(jnp.float32)
    jj = lax.broadcasted_iota(jnp.int32, (_K2_ROWS, WO), 1).astype(jnp.float32)
    def _b(v):
        return v.astype(jnp.bfloat16).astype(jnp.float32)

    gy = _b(ii / (HO - 1.0) * 2.0 - 1.0)
    gx = _b(jj / (WO - 1.0) * 2.0 - 1.0)

    def _bs(s):
        return _b(jnp.full((_K2_ROWS, WO), s, jnp.float32))

    for n in range(N):
        t00 = _bs(theta_ref[n, 0])
        t01 = _bs(theta_ref[n, 1])
        t02 = _bs(theta_ref[n, 2])
        t10 = _bs(theta_ref[n, 3])
        t11 = _bs(theta_ref[n, 4])
        t12 = _bs(theta_ref[n, 5])
        yc = (t00 * gy + t01 * gx + t02 + 1.0) * ((H - 1) / 2.0)
        xc = (t10 * gy + t11 * gx + t12 + 1.0) * ((W - 1) / 2.0)
        y0 = jnp.floor(yc)
        x0 = jnp.floor(xc)
        wy1 = 1.0 - jnp.abs(y0 + 1.0 - yc)
        wy0 = 1.0 - jnp.abs(y0 - yc)
        wx1 = 1.0 - jnp.abs(x0 + 1.0 - xc)
        wx0 = 1.0 - jnp.abs(x0 - xc)
        y0c = jnp.clip(y0, 0.0, H - 1.0).astype(jnp.int32)
        y1c = jnp.clip(y0 + 1.0, 0.0, H - 1.0).astype(jnp.int32)
        x0c = jnp.clip(x0, 0.0, W - 1.0).astype(jnp.int32)
        x1c = jnp.clip(x0 + 1.0, 0.0, W - 1.0).astype(jnp.int32)
        same = (x1c == x0c)
        wxa = jnp.where(same, wx0 + wx1, wx0)
        wxb = jnp.where(same, 0.0, wx1)
        idx_ref[0, n] = n * (H * W) + y0c * W + x0c
        idx_ref[1, n] = n * (H * W) + y1c * W + x0c
        w_ref[0, n] = wy0 * wxa
        w_ref[1, n] = wy0 * wxb
        w_ref[2, n] = wy1 * wxa
        w_ref[3, n] = wy1 * wxb


def _compute_idx_w(theta):
    return pl.pallas_call(
        _idxw_body,
        grid=(_K2_T,),
        in_specs=[pl.BlockSpec(memory_space=pltpu.SMEM)],
        out_specs=[
            pl.BlockSpec((2, N, _K2_ROWS, WO), lambda t: (0, 0, t, 0)),
            pl.BlockSpec((4, N, _K2_ROWS, WO), lambda t: (0, 0, t, 0)),
        ],
        out_shape=[
            jax.ShapeDtypeStruct((2, N, HO, WO), jnp.int32),
            jax.ShapeDtypeStruct((4, N, HO, WO), jnp.float32),
        ],
    )(theta)


_GW = 128


def _sc_gather(x_flat, idx):
    mesh = plsc.VectorSubcoreMesh(core_axis_name="c", subcore_axis_name="s")
    n_idx = idx.shape[0]

    @functools.partial(
        pl.kernel,
        out_type=jax.ShapeDtypeStruct((n_idx, CP2), jnp.float32),
        mesh=mesh,
    )
    def gather_kernel(x_hbm, i_hbm, o_hbm):
        def body(i_vmem, o_vmem):
            pltpu.sync_copy(x_hbm.at[i_vmem], o_vmem)

        pltpu.emit_pipeline(
            body,
            grid=(n_idx // _GW,),
            in_specs=[pl.BlockSpec((_GW,), lambda i: (i,))],
            out_specs=[pl.BlockSpec((_GW, CP2), lambda i: (i, 0))],
            core_axis_name=("c", "s"),
            dimension_semantics=(pltpu.PARALLEL,),
        )(i_hbm, o_hbm)

    return gather_kernel(x_flat, idx)


_K4_BP = 1024
_K4_T = P // _K4_BP


def _wsum_body(g_ref, w_ref, o_ref):
    o_ref[...] = (
        w_ref[0].reshape(_K4_BP, 1) * g_ref[0, :, 0:C]
        + w_ref[1].reshape(_K4_BP, 1) * g_ref[0, :, CP:CP + C]
        + w_ref[2].reshape(_K4_BP, 1) * g_ref[1, :, 0:C]
        + w_ref[3].reshape(_K4_BP, 1) * g_ref[1, :, CP:CP + C]
    )


def _weighted_sum(g, w_flat):
    return pl.pallas_call(
        _wsum_body,
        grid=(_K4_T,),
        in_specs=[
            pl.BlockSpec((2, _K4_BP, CP2), lambda t: (0, t, 0)),
            pl.BlockSpec((4, _K4_BP), lambda t: (0, t)),
        ],
        out_specs=pl.BlockSpec((_K4_BP, C), lambda t: (t, 0)),
        out_shape=jax.ShapeDtypeStruct((P, C), jnp.float32),
    )(g, w_flat)


def kernel(x, W_loc, b_loc):
    theta, xdup = _compute_theta(x, W_loc, b_loc)
    idx, w = _compute_idx_w(theta)
    g = _sc_gather(xdup.reshape(NHW, CP2), idx.reshape(2 * P))
    out = _weighted_sum(g.reshape(2, P, CP2), w.reshape(4, P))
    return out.reshape(N, HO, WO, C)


# speedup vs baseline: 1.6424x; 1.0806x over previous
import functools

import jax
import jax.numpy as jnp
from jax import lax
from jax.experimental import pallas as pl
from jax.experimental.pallas import tpu as pltpu
from jax.experimental.pallas import tpu_sc as plsc

N, H, W, C = 2, 384, 384, 96
HO, WO = 384, 384
P = N * HO * WO
NHW = N * H * W


_K1_ROWS = 16
_K1_T = H // _K1_ROWS


CP = 128
CP2 = 256


def _theta_body(x_ref, w_ref, b_ref, theta_ref, xpad_ref, acc_ref):
    t = pl.program_id(0)

    @pl.when(t == 0)
    def _():
        acc_ref[...] = jnp.zeros_like(acc_ref)

    xb = x_ref[...]
    acc_ref[...] += jnp.sum(xb, axis=(1, 2))
    xpad_ref[:, :, :, 0:C] = xb
    xpad_ref[:, :, :, CP:CP + C] = jnp.concatenate(
        [xb[:, :, 1:, :], xb[:, :, W - 1:, :]], axis=2)

    @pl.when(t == _K1_T - 1)
    def _():
        mean = acc_ref[...] * (1.0 / (H * W))
        theta_ref[...] = (
            jax.lax.dot(mean, w_ref[...],
                        preferred_element_type=jnp.float32)
            + b_ref[...]
        )


def _compute_theta(x, w_loc, b_loc):
    return pl.pallas_call(
        _theta_body,
        grid=(_K1_T,),
        in_specs=[
            pl.BlockSpec((N, _K1_ROWS, W, C), lambda t: (0, t, 0, 0)),
            pl.BlockSpec((C, 6), lambda t: (0, 0)),
            pl.BlockSpec((1, 6), lambda t: (0, 0)),
        ],
        out_specs=[
            pl.BlockSpec((N, 6), lambda t: (0, 0)),
            pl.BlockSpec((N, _K1_ROWS, W, CP2), lambda t: (0, t, 0, 0)),
        ],
        out_shape=[
            jax.ShapeDtypeStruct((N, 6), jnp.float32),
            jax.ShapeDtypeStruct((N, H, W, CP2), jnp.float32),
        ],
        scratch_shapes=[pltpu.VMEM((N, C), jnp.float32)],
    )(x, w_loc, b_loc.reshape(1, 6))


_K2_ROWS = 64
_K2_T = HO // _K2_ROWS


def _idxw_body(theta_ref, idx_ref, w_ref):
    t = pl.program_id(0)

    ii = (lax.broadcasted_iota(jnp.int32, (_K2_ROWS, WO), 0)
          + t * _K2_ROWS).astype(jnp.float32)
    jj = lax.broadcasted_iota(jnp.int32, (_K2_ROWS, WO), 1).astype(jnp.float32)
    def _b(v):
        return v.astype(jnp.bfloat16).astype(jnp.float32)

    gy = _b(ii / (HO - 1.0) * 2.0 - 1.0)
    gx = _b(jj / (WO - 1.0) * 2.0 - 1.0)

    def _bs(s):
        return _b(jnp.full((_K2_ROWS, WO), s, jnp.float32))

    for n in range(N):
        t00 = _bs(theta_ref[n, 0])
        t01 = _bs(theta_ref[n, 1])
        t02 = _bs(theta_ref[n, 2])
        t10 = _bs(theta_ref[n, 3])
        t11 = _bs(theta_ref[n, 4])
        t12 = _bs(theta_ref[n, 5])
        yc = (t00 * gy + t01 * gx + t02 + 1.0) * ((H - 1) / 2.0)
        xc = (t10 * gy + t11 * gx + t12 + 1.0) * ((W - 1) / 2.0)
        y0 = jnp.floor(yc)
        x0 = jnp.floor(xc)
        wy1 = 1.0 - jnp.abs(y0 + 1.0 - yc)
        wy0 = 1.0 - jnp.abs(y0 - yc)
        wx1 = 1.0 - jnp.abs(x0 + 1.0 - xc)
        wx0 = 1.0 - jnp.abs(x0 - xc)
        y0c = jnp.clip(y0, 0.0, H - 1.0).astype(jnp.int32)
        y1c = jnp.clip(y0 + 1.0, 0.0, H - 1.0).astype(jnp.int32)
        x0c = jnp.clip(x0, 0.0, W - 1.0).astype(jnp.int32)
        x1c = jnp.clip(x0 + 1.0, 0.0, W - 1.0).astype(jnp.int32)
        same = (x1c == x0c)
        wxa = jnp.where(same, wx0 + wx1, wx0)
        wxb = jnp.where(same, 0.0, wx1)
        idx_ref[0, n] = n * (H * W) + y0c * W + x0c
        idx_ref[1, n] = n * (H * W) + y1c * W + x0c
        w_ref[0, n] = wy0 * wxa
        w_ref[1, n] = wy0 * wxb
        w_ref[2, n] = wy1 * wxa
        w_ref[3, n] = wy1 * wxb


def _compute_idx_w(theta):
    return pl.pallas_call(
        _idxw_body,
        grid=(_K2_T,),
        in_specs=[pl.BlockSpec(memory_space=pltpu.SMEM)],
        out_specs=[
            pl.BlockSpec((2, N, _K2_ROWS, WO), lambda t: (0, 0, t, 0)),
            pl.BlockSpec((4, N, _K2_ROWS, WO), lambda t: (0, 0, t, 0)),
        ],
        out_shape=[
            jax.ShapeDtypeStruct((2, N, HO, WO), jnp.int32),
            jax.ShapeDtypeStruct((4, N, HO, WO), jnp.float32),
        ],
    )(theta)


_GW = 128


def _sc_gather(x_flat, idx):
    mesh = plsc.VectorSubcoreMesh(core_axis_name="c", subcore_axis_name="s")
    n_idx = idx.shape[0] * idx.shape[1]

    @functools.partial(
        pl.kernel,
        out_type=jax.ShapeDtypeStruct((n_idx, CP2), jnp.float32),
        mesh=mesh,
    )
    def gather_kernel(x_hbm, i_hbm, o_hbm):
        def body(i_vmem, o_vmem):
            pltpu.sync_copy(x_hbm.at[i_vmem.at[0]], o_vmem)

        pltpu.emit_pipeline(
            body,
            grid=(n_idx // _GW,),
            in_specs=[pl.BlockSpec((1, _GW), lambda i: (i, 0))],
            out_specs=[pl.BlockSpec((_GW, CP2), lambda i: (i, 0))],
            core_axis_name=("c", "s"),
            dimension_semantics=(pltpu.PARALLEL,),
        )(i_hbm, o_hbm)

    return gather_kernel(x_flat, idx)


_K4_ROWS = 8
_K4_T = HO // _K4_ROWS


def _wsum_body(g_ref, w_ref, o_ref):
    o_ref[0] = (
        w_ref[0, 0][..., None] * g_ref[0, 0, :, :, 0:C]
        + w_ref[1, 0][..., None] * g_ref[0, 0, :, :, CP:CP + C]
        + w_ref[2, 0][..., None] * g_ref[1, 0, :, :, 0:C]
        + w_ref[3, 0][..., None] * g_ref[1, 0, :, :, CP:CP + C]
    )


def _weighted_sum(g, w):
    return pl.pallas_call(
        _wsum_body,
        grid=(N, _K4_T),
        in_specs=[
            pl.BlockSpec((2, 1, _K4_ROWS, WO, CP2), lambda n, t: (0, n, t, 0, 0)),
            pl.BlockSpec((4, 1, _K4_ROWS, WO), lambda n, t: (0, n, t, 0)),
        ],
        out_specs=pl.BlockSpec((1, _K4_ROWS, WO, C), lambda n, t: (n, t, 0, 0)),
        out_shape=jax.ShapeDtypeStruct((N, HO, WO, C), jnp.float32),
    )(g, w)


def kernel(x, W_loc, b_loc):
    theta, xdup = _compute_theta(x, W_loc, b_loc)
    idx, w = _compute_idx_w(theta)
    g = _sc_gather(xdup.reshape(NHW, CP2), idx.reshape(2 * P // _GW, _GW))
    return _weighted_sum(g.reshape(2, N, HO, WO, CP2), w)
